# baseline (speedup 1.0000x reference)
import functools

import jax
import jax.numpy as jnp
import numpy as np
from jax import lax
from jax.experimental import pallas as pl
from jax.experimental.pallas import tpu as pltpu
from jax.experimental.pallas import tpu_sc as plsc

NUM_USERS = 25000
NUM_ITEMS = 25000
N = NUM_USERS + NUM_ITEMS
E = 1600000
D = 32
B = 4096

NC = 2
NS = 16
NW = NC * NS

CK = 500
SK = 250
CHUNKS = E // CK
SCHUNKS = E // SK
TILE_CHUNKS = CHUNKS // NW
SUP = 4
SUPERS = TILE_CHUNKS // SUP
PAIRS = SUP // 2

_FULL = SK // 16
_REM = SK - _FULL * 16

SROWS = SK
NSTRIPES = N // SROWS
STRIPE_STEPS = (NSTRIPES + NS - 1) // NS

_PERM = np.arange(D).reshape(2, D // 2).T.reshape(-1)

_ILV = plsc.PackFormat.INTERLEAVED

_mesh = plsc.VectorSubcoreMesh(core_axis_name="c", subcore_axis_name="s")
_params = pltpu.CompilerParams(use_tc_tiling_on_sc=False, needs_layout_passes=False)


@functools.partial(
    pl.kernel,
    out_type=jax.ShapeDtypeStruct((NC, N, D), jnp.float32),
    mesh=_mesh,
    compiler_params=_params,
    scratch_types=[
        pltpu.VMEM_SHARED((N, D), jnp.float32),
        pltpu.VMEM((SUP, CK), jnp.int32),
        pltpu.VMEM((SUP * 2, SK), jnp.int32),
        pltpu.VMEM((SUP, CK), jnp.float32),
        pltpu.VMEM((CK, D), jnp.bfloat16),
        pltpu.VMEM((CK, D), jnp.bfloat16),
        pltpu.VMEM((SK, D), jnp.float32),
        pltpu.SemaphoreType.DMA,
        pltpu.SemaphoreType.DMA,
    ],
)
def _spmv(table, src2d, dst2d, w2d, out, acc, srcb, dstb, wb, g0, g1, sc0,
          sem0, sem1):
    cid = lax.axis_index("c")
    sid = lax.axis_index("s")
    tile = cid * NS + sid

    zeros16 = jnp.zeros((16,), jnp.float32)

    def _zfill(r, _):
        sc0[r, pl.ds(0, 16)] = zeros16
        sc0[r, pl.ds(16, 16)] = zeros16
        return _

    lax.fori_loop(0, SROWS, _zfill, 0)

    def _zcopy(k, carry):
        c = sid + k * NS

        @pl.when(c < NSTRIPES)
        def _zc():
            pltpu.sync_copy(sc0, acc.at[pl.ds(c * SROWS, SROWS)])

        return carry

    lax.fori_loop(0, STRIPE_STEPS, _zcopy, 0)
    plsc.subcore_barrier()

    chunk0 = tile * TILE_CHUNKS

    def _scale_scatter(g, c, h):
        def _grp(gi, carry):
            wvec = wb[c, pl.ds(h * SK + gi * 16, 16)]
            for lane in range(16):
                e = gi * 16 + lane
                lo, hi = plsc.unpack(g[h * SK + e], format=_ILV)
                wv = wvec[lane]
                sc0[e, pl.ds(0, 16)] = lo * wv
                sc0[e, pl.ds(16, 16)] = hi * wv
            return carry

        lax.fori_loop(0, _FULL, _grp, 0)
        wvec = wb[c, pl.ds(h * SK + SK - 16, 16)]
        for lane in range(16 - _REM, 16):
            e = SK - 16 + lane
            lo, hi = plsc.unpack(g[h * SK + e], format=_ILV)
            wv = wvec[lane]
            sc0[e, pl.ds(0, 16)] = lo * wv
            sc0[e, pl.ds(16, 16)] = hi * wv
        pltpu.sync_copy(sc0, acc.at[dstb.at[c * 2 + h]], add=True)

    def _super(s, _):
        base = chunk0 + s * SUP
        pltpu.sync_copy(src2d.at[pl.ds(base, SUP)], srcb)
        pltpu.sync_copy(dst2d.at[pl.ds(base * 2, SUP * 2)], dstb)
        pltpu.sync_copy(w2d.at[pl.ds(base, SUP)], wb)

        def _pair(pc, carry):
            c0 = pc * 2

            @pl.when(pc == 0)
            def _prime():
                pltpu.async_copy(table.at[srcb.at[0]], g0, sem0)

            pltpu.make_async_copy(table.at[pl.ds(0, CK)], g0, sem0).wait()
            pltpu.async_copy(table.at[srcb.at[c0 + 1]], g1, sem1)
            _scale_scatter(g0, c0, 0)
            _scale_scatter(g0, c0, 1)

            pltpu.make_async_copy(table.at[pl.ds(0, CK)], g1, sem1).wait()

            @pl.when(pc + 1 < PAIRS)
            def _next():
                pltpu.async_copy(table.at[srcb.at[c0 + 2]], g0, sem0)

            _scale_scatter(g1, c0 + 1, 0)
            _scale_scatter(g1, c0 + 1, 1)
            return carry

        lax.fori_loop(0, PAIRS, _pair, 0)
        return _

    lax.fori_loop(0, SUPERS, _super, 0)
    plsc.subcore_barrier()

    def _wcopy(k, carry):
        c = sid + k * NS

        @pl.when(c < NSTRIPES)
        def _wc():
            pltpu.sync_copy(acc.at[pl.ds(c * SROWS, SROWS)],
                            out.at[cid, pl.ds(c * SROWS, SROWS)])

        return carry

    lax.fori_loop(0, STRIPE_STEPS, _wcopy, 0)


CROWS = 400
CCHUNKS = N // CROWS
CSTEPS = (CCHUNKS + NW - 1) // NW


@functools.partial(
    pl.kernel,
    out_type=jax.ShapeDtypeStruct((N, D), jnp.bfloat16),
    mesh=_mesh,
    compiler_params=_params,
    scratch_types=[
        pltpu.VMEM((CROWS, D), jnp.float32),
        pltpu.VMEM((CROWS, D), jnp.float32),
        pltpu.VMEM((CROWS, D), jnp.bfloat16),
    ],
)
def _combine(p, out, a, b, o):
    cid = lax.axis_index("c")
    sid = lax.axis_index("s")
    tile = cid * NS + sid

    def _step(k, carry):
        c = tile + k * NW

        @pl.when(c < CCHUNKS)
        def _body():
            r0 = c * CROWS
            pltpu.sync_copy(p.at[0, pl.ds(r0, CROWS)], a)
            pltpu.sync_copy(p.at[1, pl.ds(r0, CROWS)], b)

            def _add(r, _):
                lo = a[r, pl.ds(0, 16)] + b[r, pl.ds(0, 16)]
                hi = a[r, pl.ds(16, 16)] + b[r, pl.ds(16, 16)]
                o[r] = plsc.pack(lo, hi, format=_ILV)
                return _

            lax.fori_loop(0, CROWS, _add, 0)
            pltpu.sync_copy(o, out.at[pl.ds(r0, CROWS)])

        return carry

    lax.fori_loop(0, CSTEPS, _step, 0)


PPT = B // NW


@functools.partial(
    pl.kernel,
    out_type=jax.ShapeDtypeStruct((B,), jnp.float32),
    mesh=_mesh,
    compiler_params=_params,
    scratch_types=[
        pltpu.VMEM((PPT,), jnp.int32),
        pltpu.VMEM((PPT,), jnp.int32),
        pltpu.VMEM((PPT,), jnp.int32),
        pltpu.VMEM((PPT, D), jnp.float32),
        pltpu.VMEM((PPT, D), jnp.float32),
        pltpu.VMEM((3, PPT, D), jnp.bfloat16),
        pltpu.VMEM((3, PPT, D), jnp.bfloat16),
        pltpu.VMEM((PPT,), jnp.float32),
        pltpu.SemaphoreType.DMA,
    ],
)
def _final(users, items, uemb, iemb, t1, t2, t3, out,
           uidx, iidx, iidx2, gu, gi, tu, ti, obuf, sem):
    cid = lax.axis_index("c")
    sid = lax.axis_index("s")
    tile = cid * NS + sid
    base = tile * PPT

    pltpu.sync_copy(users.at[pl.ds(base, PPT)], uidx)
    pltpu.sync_copy(items.at[pl.ds(base, PPT)], iidx)

    off = jnp.full((16,), NUM_USERS, jnp.int32)

    def _shift(v, _):
        iidx2[pl.ds(v * 16, 16)] = iidx[pl.ds(v * 16, 16)] + off
        return _

    lax.fori_loop(0, PPT // 16, _shift, 0)

    pltpu.async_copy(uemb.at[uidx], gu, sem).wait()
    pltpu.async_copy(iemb.at[iidx], gi, sem).wait()
    pltpu.async_copy(t1.at[uidx], tu.at[0], sem).wait()
    pltpu.async_copy(t2.at[uidx], tu.at[1], sem).wait()
    pltpu.async_copy(t3.at[uidx], tu.at[2], sem).wait()
    pltpu.async_copy(t1.at[iidx2], ti.at[0], sem).wait()
    pltpu.async_copy(t2.at[iidx2], ti.at[1], sem).wait()
    pltpu.async_copy(t3.at[iidx2], ti.at[2], sem).wait()

    lane_iota = lax.iota(jnp.int32, 16)

    def _group(gp, carry):
        res = jnp.zeros((16,), jnp.float32)
        for lane in range(16):
            p_ = gp * 16 + lane
            su_lo = gu[p_, pl.ds(0, 16)]
            su_hi = gu[p_, pl.ds(16, 16)]
            si_lo = gi[p_, pl.ds(0, 16)]
            si_hi = gi[p_, pl.ds(16, 16)]
            for k in range(3):
                ulo, uhi = plsc.unpack(tu[k, p_], format=_ILV)
                ilo, ihi = plsc.unpack(ti[k, p_], format=_ILV)
                su_lo = su_lo + ulo
                su_hi = su_hi + uhi
                si_lo = si_lo + ilo
                si_hi = si_hi + ihi
            s = jnp.sum(su_lo * si_lo + su_hi * si_hi) * 0.0625
            res = jnp.where(lane_iota == lane, s, res)
        obuf[pl.ds(gp * 16, 16)] = res
        return carry

    lax.fori_loop(0, PPT // 16, _group, 0)
    pltpu.sync_copy(obuf, out.at[pl.ds(base, PPT)])


def kernel(users, items, edge_index, edge_weight, user_emb, item_emb):
    src2d = edge_index[0].reshape(CHUNKS, CK)
    dst2d = edge_index[1].reshape(SCHUNKS, SK)
    w2d = edge_weight.reshape(CHUNKS, CK)
    tab0 = jnp.concatenate([user_emb, item_emb], axis=0)
    tab0 = tab0[:, _PERM].astype(jnp.bfloat16)

    p1 = _spmv(tab0, src2d, dst2d, w2d)
    t1 = _combine(p1)
    p2 = _spmv(t1, src2d, dst2d, w2d)
    t2 = _combine(p2)
    p3 = _spmv(t2, src2d, dst2d, w2d)
    t3 = _combine(p3)
    return _final(users, items, user_emb, item_emb, t1, t2, t3)


# speedup vs baseline: 1.2019x; 1.2019x over previous
import functools

import jax
import jax.numpy as jnp
import numpy as np
from jax import lax
from jax.experimental import pallas as pl
from jax.experimental.pallas import tpu as pltpu
from jax.experimental.pallas import tpu_sc as plsc

NUM_USERS = 25000
NUM_ITEMS = 25000
N = NUM_USERS + NUM_ITEMS
E = 1600000
D = 32
B = 4096

NC = 2
NS = 16
NW = NC * NS

CK = 500
SK = 250
CHUNKS = E // CK
SCHUNKS = E // SK
TILE_CHUNKS = CHUNKS // NW
SUP = 4
SUPERS = TILE_CHUNKS // SUP
PAIRS = SUP // 2

_FULL = SK // 16
_REM = SK - _FULL * 16

SROWS = SK
NSTRIPES = N // SROWS
STRIPE_STEPS = (NSTRIPES + NS - 1) // NS

_PERM = np.arange(D).reshape(2, D // 2).T.reshape(-1)

_ILV = plsc.PackFormat.INTERLEAVED

_mesh = plsc.VectorSubcoreMesh(core_axis_name="c", subcore_axis_name="s")
_params = pltpu.CompilerParams(use_tc_tiling_on_sc=False, needs_layout_passes=False)


@functools.partial(
    pl.kernel,
    out_type=jax.ShapeDtypeStruct((NC, N, D), jnp.float32),
    mesh=_mesh,
    compiler_params=_params,
    scratch_types=[
        pltpu.VMEM_SHARED((N, D), jnp.float32),
        pltpu.VMEM((SUP, CK), jnp.int32),
        pltpu.VMEM((SUP * 2, SK), jnp.int32),
        pltpu.VMEM((SUP, CK), jnp.float32),
        pltpu.VMEM((CK, D), jnp.bfloat16),
        pltpu.VMEM((CK, D), jnp.bfloat16),
        pltpu.VMEM((SK, D), jnp.float32),
        pltpu.SemaphoreType.DMA,
        pltpu.SemaphoreType.DMA,
    ],
)
def _spmv(table, src2d, dst2d, w2d, out, acc, srcb, dstb, wb, g0, g1, sc0,
          sem0, sem1):
    cid = lax.axis_index("c")
    sid = lax.axis_index("s")
    tile = cid * NS + sid

    zeros16 = jnp.zeros((16,), jnp.float32)

    def _zfill(r, _):
        sc0[r, pl.ds(0, 16)] = zeros16
        sc0[r, pl.ds(16, 16)] = zeros16
        return _

    lax.fori_loop(0, SROWS, _zfill, 0)

    def _zcopy(k, carry):
        c = sid + k * NS

        @pl.when(c < NSTRIPES)
        def _zc():
            pltpu.sync_copy(sc0, acc.at[pl.ds(c * SROWS, SROWS)])

        return carry

    lax.fori_loop(0, STRIPE_STEPS, _zcopy, 0)
    plsc.subcore_barrier()

    chunk0 = tile * TILE_CHUNKS

    def _scale_scatter(g, c, h):
        for gi in range(_FULL):
            wvec = wb[c, pl.ds(h * SK + gi * 16, 16)]
            for lane in range(16):
                e = gi * 16 + lane
                lo, hi = plsc.unpack(g[h * SK + e], format=_ILV)
                wv = wvec[lane]
                sc0[e, pl.ds(0, 16)] = lo * wv
                sc0[e, pl.ds(16, 16)] = hi * wv
        wvec = wb[c, pl.ds(h * SK + SK - 16, 16)]
        for lane in range(16 - _REM, 16):
            e = SK - 16 + lane
            lo, hi = plsc.unpack(g[h * SK + e], format=_ILV)
            wv = wvec[lane]
            sc0[e, pl.ds(0, 16)] = lo * wv
            sc0[e, pl.ds(16, 16)] = hi * wv
        pltpu.sync_copy(sc0, acc.at[dstb.at[c * 2 + h]], add=True)

    def _super(s, _):
        base = chunk0 + s * SUP
        pltpu.sync_copy(src2d.at[pl.ds(base, SUP)], srcb)
        pltpu.sync_copy(dst2d.at[pl.ds(base * 2, SUP * 2)], dstb)
        pltpu.sync_copy(w2d.at[pl.ds(base, SUP)], wb)

        def _pair(pc, carry):
            c0 = pc * 2

            @pl.when(pc == 0)
            def _prime():
                pltpu.async_copy(table.at[srcb.at[0]], g0, sem0)

            pltpu.make_async_copy(table.at[pl.ds(0, CK)], g0, sem0).wait()
            pltpu.async_copy(table.at[srcb.at[c0 + 1]], g1, sem1)
            _scale_scatter(g0, c0, 0)
            _scale_scatter(g0, c0, 1)

            pltpu.make_async_copy(table.at[pl.ds(0, CK)], g1, sem1).wait()

            @pl.when(pc + 1 < PAIRS)
            def _next():
                pltpu.async_copy(table.at[srcb.at[c0 + 2]], g0, sem0)

            _scale_scatter(g1, c0 + 1, 0)
            _scale_scatter(g1, c0 + 1, 1)
            return carry

        lax.fori_loop(0, PAIRS, _pair, 0)
        return _

    lax.fori_loop(0, SUPERS, _super, 0)
    plsc.subcore_barrier()

    def _wcopy(k, carry):
        c = sid + k * NS

        @pl.when(c < NSTRIPES)
        def _wc():
            pltpu.sync_copy(acc.at[pl.ds(c * SROWS, SROWS)],
                            out.at[cid, pl.ds(c * SROWS, SROWS)])

        return carry

    lax.fori_loop(0, STRIPE_STEPS, _wcopy, 0)


CROWS = 400
CCHUNKS = N // CROWS
CSTEPS = (CCHUNKS + NW - 1) // NW


@functools.partial(
    pl.kernel,
    out_type=jax.ShapeDtypeStruct((N, D), jnp.bfloat16),
    mesh=_mesh,
    compiler_params=_params,
    scratch_types=[
        pltpu.VMEM((CROWS, D), jnp.float32),
        pltpu.VMEM((CROWS, D), jnp.float32),
        pltpu.VMEM((CROWS, D), jnp.bfloat16),
    ],
)
def _combine(p, out, a, b, o):
    cid = lax.axis_index("c")
    sid = lax.axis_index("s")
    tile = cid * NS + sid

    def _step(k, carry):
        c = tile + k * NW

        @pl.when(c < CCHUNKS)
        def _body():
            r0 = c * CROWS
            pltpu.sync_copy(p.at[0, pl.ds(r0, CROWS)], a)
            pltpu.sync_copy(p.at[1, pl.ds(r0, CROWS)], b)

            def _add(r, _):
                lo = a[r, pl.ds(0, 16)] + b[r, pl.ds(0, 16)]
                hi = a[r, pl.ds(16, 16)] + b[r, pl.ds(16, 16)]
                o[r] = plsc.pack(lo, hi, format=_ILV)
                return _

            lax.fori_loop(0, CROWS, _add, 0)
            pltpu.sync_copy(o, out.at[pl.ds(r0, CROWS)])

        return carry

    lax.fori_loop(0, CSTEPS, _step, 0)


PPT = B // NW


@functools.partial(
    pl.kernel,
    out_type=jax.ShapeDtypeStruct((B,), jnp.float32),
    mesh=_mesh,
    compiler_params=_params,
    scratch_types=[
        pltpu.VMEM((PPT,), jnp.int32),
        pltpu.VMEM((PPT,), jnp.int32),
        pltpu.VMEM((PPT,), jnp.int32),
        pltpu.VMEM((PPT, D), jnp.float32),
        pltpu.VMEM((PPT, D), jnp.float32),
        pltpu.VMEM((3, PPT, D), jnp.bfloat16),
        pltpu.VMEM((3, PPT, D), jnp.bfloat16),
        pltpu.VMEM((PPT,), jnp.float32),
        pltpu.SemaphoreType.DMA,
    ],
)
def _final(users, items, uemb, iemb, t1, t2, t3, out,
           uidx, iidx, iidx2, gu, gi, tu, ti, obuf, sem):
    cid = lax.axis_index("c")
    sid = lax.axis_index("s")
    tile = cid * NS + sid
    base = tile * PPT

    pltpu.sync_copy(users.at[pl.ds(base, PPT)], uidx)
    pltpu.sync_copy(items.at[pl.ds(base, PPT)], iidx)

    off = jnp.full((16,), NUM_USERS, jnp.int32)

    def _shift(v, _):
        iidx2[pl.ds(v * 16, 16)] = iidx[pl.ds(v * 16, 16)] + off
        return _

    lax.fori_loop(0, PPT // 16, _shift, 0)

    pltpu.async_copy(uemb.at[uidx], gu, sem).wait()
    pltpu.async_copy(iemb.at[iidx], gi, sem).wait()
    pltpu.async_copy(t1.at[uidx], tu.at[0], sem).wait()
    pltpu.async_copy(t2.at[uidx], tu.at[1], sem).wait()
    pltpu.async_copy(t3.at[uidx], tu.at[2], sem).wait()
    pltpu.async_copy(t1.at[iidx2], ti.at[0], sem).wait()
    pltpu.async_copy(t2.at[iidx2], ti.at[1], sem).wait()
    pltpu.async_copy(t3.at[iidx2], ti.at[2], sem).wait()

    lane_iota = lax.iota(jnp.int32, 16)

    def _group(gp, carry):
        res = jnp.zeros((16,), jnp.float32)
        for lane in range(16):
            p_ = gp * 16 + lane
            su_lo = gu[p_, pl.ds(0, 16)]
            su_hi = gu[p_, pl.ds(16, 16)]
            si_lo = gi[p_, pl.ds(0, 16)]
            si_hi = gi[p_, pl.ds(16, 16)]
            for k in range(3):
                ulo, uhi = plsc.unpack(tu[k, p_], format=_ILV)
                ilo, ihi = plsc.unpack(ti[k, p_], format=_ILV)
                su_lo = su_lo + ulo
                su_hi = su_hi + uhi
                si_lo = si_lo + ilo
                si_hi = si_hi + ihi
            s = jnp.sum(su_lo * si_lo + su_hi * si_hi) * 0.0625
            res = jnp.where(lane_iota == lane, s, res)
        obuf[pl.ds(gp * 16, 16)] = res
        return carry

    lax.fori_loop(0, PPT // 16, _group, 0)
    pltpu.sync_copy(obuf, out.at[pl.ds(base, PPT)])


def kernel(users, items, edge_index, edge_weight, user_emb, item_emb):
    src2d = edge_index[0].reshape(CHUNKS, CK)
    dst2d = edge_index[1].reshape(SCHUNKS, SK)
    w2d = edge_weight.reshape(CHUNKS, CK)
    tab0 = jnp.concatenate([user_emb, item_emb], axis=0)
    tab0 = tab0[:, _PERM].astype(jnp.bfloat16)

    p1 = _spmv(tab0, src2d, dst2d, w2d)
    t1 = _combine(p1)
    p2 = _spmv(t1, src2d, dst2d, w2d)
    t2 = _combine(p2)
    p3 = _spmv(t2, src2d, dst2d, w2d)
    t3 = _combine(p3)
    return _final(users, items, user_emb, item_emb, t1, t2, t3)


# speedup vs baseline: 1.5301x; 1.2730x over previous
import functools

import jax
import jax.numpy as jnp
import numpy as np
from jax import lax
from jax.experimental import pallas as pl
from jax.experimental.pallas import tpu as pltpu
from jax.experimental.pallas import tpu_sc as plsc

NUM_USERS = 25000
NUM_ITEMS = 25000
N = NUM_USERS + NUM_ITEMS
E = 1600000
D = 32
B = 4096

NC = 2
NS = 16
NW = NC * NS

CK = 250
CHUNKS = E // CK
TILE_CHUNKS = CHUNKS // NW
SUP = 8
SUPERS = TILE_CHUNKS // SUP
PAIRS = SUP // 2

_FULL = CK // 16
_REM = CK - _FULL * 16
_WGROUPS = tuple((i * 16, tuple(range(16))) for i in range(_FULL))
if _REM:
    _WGROUPS += ((CK - 16, tuple(range(16 - _REM, 16))),)

SROWS = 200
NSTRIPES = N // SROWS
STRIPE_STEPS = (NSTRIPES + NS - 1) // NS

_PERM = np.arange(D).reshape(2, D // 2).T.reshape(-1)

_ILV = plsc.PackFormat.INTERLEAVED

_mesh = plsc.VectorSubcoreMesh(core_axis_name="c", subcore_axis_name="s")
_params = pltpu.CompilerParams(use_tc_tiling_on_sc=False, needs_layout_passes=False)


@functools.partial(
    pl.kernel,
    out_type=jax.ShapeDtypeStruct((NC, N, D), jnp.float32),
    mesh=_mesh,
    compiler_params=_params,
    scratch_types=[
        pltpu.VMEM_SHARED((N, D), jnp.float32),
        pltpu.VMEM((SUP, CK), jnp.int32),
        pltpu.VMEM((SUP, CK), jnp.int32),
        pltpu.VMEM((SUP, CK), jnp.float32),
        pltpu.VMEM((CK, D), jnp.bfloat16),
        pltpu.VMEM((CK, D), jnp.bfloat16),
        pltpu.VMEM((CK, D), jnp.float32),
        pltpu.VMEM((SROWS, D), jnp.float32),
        pltpu.SemaphoreType.DMA,
        pltpu.SemaphoreType.DMA,
    ],
)
def _spmv(table, src2d, dst2d, w2d, out, acc, srcb, dstb, wb, g0, g1, sc0,
          zbuf, sem0, sem1):
    cid = lax.axis_index("c")
    sid = lax.axis_index("s")
    tile = cid * NS + sid

    zeros16 = jnp.zeros((16,), jnp.float32)

    def _zfill(r, _):
        zbuf[r, pl.ds(0, 16)] = zeros16
        zbuf[r, pl.ds(16, 16)] = zeros16
        return _

    lax.fori_loop(0, SROWS, _zfill, 0)

    def _zcopy(k, carry):
        c = sid + k * NS

        @pl.when(c < NSTRIPES)
        def _zc():
            pltpu.sync_copy(zbuf, acc.at[pl.ds(c * SROWS, SROWS)])

        return carry

    lax.fori_loop(0, STRIPE_STEPS, _zcopy, 0)
    plsc.subcore_barrier()

    chunk0 = tile * TILE_CHUNKS

    def _scale(g, sc, c):
        for goff, glanes in _WGROUPS:
            wvec = wb[c, pl.ds(goff, 16)]
            for lane in glanes:
                e = goff + lane
                lo, hi = plsc.unpack(g[e], format=_ILV)
                wv = wvec[lane]
                sc[e, pl.ds(0, 16)] = lo * wv
                sc[e, pl.ds(16, 16)] = hi * wv

    def _super(s, _):
        base = chunk0 + s * SUP
        pltpu.sync_copy(src2d.at[pl.ds(base, SUP)], srcb)
        pltpu.sync_copy(dst2d.at[pl.ds(base, SUP)], dstb)
        pltpu.sync_copy(w2d.at[pl.ds(base, SUP)], wb)

        def _pair(pc, carry):
            c0 = pc * 2

            @pl.when(pc == 0)
            def _prime():
                pltpu.async_copy(table.at[srcb.at[0]], g0, sem0)

            pltpu.make_async_copy(table.at[pl.ds(0, CK)], g0, sem0).wait()
            pltpu.async_copy(table.at[srcb.at[c0 + 1]], g1, sem1)
            _scale(g0, sc0, c0)
            pltpu.sync_copy(sc0, acc.at[dstb.at[c0]], add=True)

            pltpu.make_async_copy(table.at[pl.ds(0, CK)], g1, sem1).wait()

            @pl.when(pc + 1 < PAIRS)
            def _next():
                pltpu.async_copy(table.at[srcb.at[c0 + 2]], g0, sem0)

            _scale(g1, sc0, c0 + 1)
            pltpu.sync_copy(sc0, acc.at[dstb.at[c0 + 1]], add=True)
            return carry

        lax.fori_loop(0, PAIRS, _pair, 0)
        return _

    lax.fori_loop(0, SUPERS, _super, 0)
    plsc.subcore_barrier()

    def _wcopy(k, carry):
        c = sid + k * NS

        @pl.when(c < NSTRIPES)
        def _wc():
            pltpu.sync_copy(acc.at[pl.ds(c * SROWS, SROWS)],
                            out.at[cid, pl.ds(c * SROWS, SROWS)])

        return carry

    lax.fori_loop(0, STRIPE_STEPS, _wcopy, 0)


CROWS = 400
CCHUNKS = N // CROWS
CSTEPS = (CCHUNKS + NW - 1) // NW


@functools.partial(
    pl.kernel,
    out_type=jax.ShapeDtypeStruct((N, D), jnp.bfloat16),
    mesh=_mesh,
    compiler_params=_params,
    scratch_types=[
        pltpu.VMEM((CROWS, D), jnp.float32),
        pltpu.VMEM((CROWS, D), jnp.float32),
        pltpu.VMEM((CROWS, D), jnp.bfloat16),
    ],
)
def _combine(p, out, a, b, o):
    cid = lax.axis_index("c")
    sid = lax.axis_index("s")
    tile = cid * NS + sid

    def _step(k, carry):
        c = tile + k * NW

        @pl.when(c < CCHUNKS)
        def _body():
            r0 = c * CROWS
            pltpu.sync_copy(p.at[0, pl.ds(r0, CROWS)], a)
            pltpu.sync_copy(p.at[1, pl.ds(r0, CROWS)], b)

            def _add(r, _):
                lo = a[r, pl.ds(0, 16)] + b[r, pl.ds(0, 16)]
                hi = a[r, pl.ds(16, 16)] + b[r, pl.ds(16, 16)]
                o[r] = plsc.pack(lo, hi, format=_ILV)
                return _

            lax.fori_loop(0, CROWS, _add, 0)
            pltpu.sync_copy(o, out.at[pl.ds(r0, CROWS)])

        return carry

    lax.fori_loop(0, CSTEPS, _step, 0)


PPT = B // NW


@functools.partial(
    pl.kernel,
    out_type=jax.ShapeDtypeStruct((B,), jnp.float32),
    mesh=_mesh,
    compiler_params=_params,
    scratch_types=[
        pltpu.VMEM((PPT,), jnp.int32),
        pltpu.VMEM((PPT,), jnp.int32),
        pltpu.VMEM((PPT,), jnp.int32),
        pltpu.VMEM((PPT, D), jnp.float32),
        pltpu.VMEM((PPT, D), jnp.float32),
        pltpu.VMEM((2, PPT, D), jnp.bfloat16),
        pltpu.VMEM((2, PPT, D), jnp.bfloat16),
        pltpu.VMEM((2, PPT, D), jnp.float32),
        pltpu.VMEM((2, PPT, D), jnp.float32),
        pltpu.VMEM((PPT,), jnp.float32),
        pltpu.SemaphoreType.DMA,
    ],
)
def _final(users, items, uemb, iemb, t1, t2, p3, out,
           uidx, iidx, iidx2, gu, gi, tu, ti, pu, pi, obuf, sem):
    cid = lax.axis_index("c")
    sid = lax.axis_index("s")
    tile = cid * NS + sid
    base = tile * PPT

    pltpu.sync_copy(users.at[pl.ds(base, PPT)], uidx)
    pltpu.sync_copy(items.at[pl.ds(base, PPT)], iidx)

    off = jnp.full((16,), NUM_USERS, jnp.int32)

    def _shift(v, _):
        iidx2[pl.ds(v * 16, 16)] = iidx[pl.ds(v * 16, 16)] + off
        return _

    lax.fori_loop(0, PPT // 16, _shift, 0)

    pltpu.async_copy(uemb.at[uidx], gu, sem).wait()
    pltpu.async_copy(iemb.at[iidx], gi, sem).wait()
    pltpu.async_copy(t1.at[uidx], tu.at[0], sem).wait()
    pltpu.async_copy(t2.at[uidx], tu.at[1], sem).wait()
    pltpu.async_copy(t1.at[iidx2], ti.at[0], sem).wait()
    pltpu.async_copy(t2.at[iidx2], ti.at[1], sem).wait()
    pltpu.async_copy(p3.at[0].at[uidx], pu.at[0], sem).wait()
    pltpu.async_copy(p3.at[1].at[uidx], pu.at[1], sem).wait()
    pltpu.async_copy(p3.at[0].at[iidx2], pi.at[0], sem).wait()
    pltpu.async_copy(p3.at[1].at[iidx2], pi.at[1], sem).wait()

    lane_iota = lax.iota(jnp.int32, 16)

    def _group(gp, carry):
        res = jnp.zeros((16,), jnp.float32)
        for lane in range(16):
            p_ = gp * 16 + lane
            su_lo = gu[p_, pl.ds(0, 16)] + pu[0, p_, pl.ds(0, 16)] + pu[1, p_, pl.ds(0, 16)]
            su_hi = gu[p_, pl.ds(16, 16)] + pu[0, p_, pl.ds(16, 16)] + pu[1, p_, pl.ds(16, 16)]
            si_lo = gi[p_, pl.ds(0, 16)] + pi[0, p_, pl.ds(0, 16)] + pi[1, p_, pl.ds(0, 16)]
            si_hi = gi[p_, pl.ds(16, 16)] + pi[0, p_, pl.ds(16, 16)] + pi[1, p_, pl.ds(16, 16)]
            for k in range(2):
                ulo, uhi = plsc.unpack(tu[k, p_], format=_ILV)
                ilo, ihi = plsc.unpack(ti[k, p_], format=_ILV)
                su_lo = su_lo + ulo
                su_hi = su_hi + uhi
                si_lo = si_lo + ilo
                si_hi = si_hi + ihi
            s = jnp.sum(su_lo * si_lo + su_hi * si_hi) * 0.0625
            res = jnp.where(lane_iota == lane, s, res)
        obuf[pl.ds(gp * 16, 16)] = res
        return carry

    lax.fori_loop(0, PPT // 16, _group, 0)
    pltpu.sync_copy(obuf, out.at[pl.ds(base, PPT)])


def kernel(users, items, edge_index, edge_weight, user_emb, item_emb):
    src2d = edge_index[0].reshape(CHUNKS, CK)
    dst2d = edge_index[1].reshape(CHUNKS, CK)
    w2d = edge_weight.reshape(CHUNKS, CK)
    tab0 = jnp.concatenate([user_emb, item_emb], axis=0)
    tab0 = tab0[:, _PERM].astype(jnp.bfloat16)

    p1 = _spmv(tab0, src2d, dst2d, w2d)
    t1 = _combine(p1)
    p2 = _spmv(t1, src2d, dst2d, w2d)
    t2 = _combine(p2)
    p3 = _spmv(t2, src2d, dst2d, w2d)
    return _final(users, items, user_emb, item_emb, t1, t2, p3)


# speedup vs baseline: 1.5327x; 1.0017x over previous
import functools

import jax
import jax.numpy as jnp
import numpy as np
from jax import lax
from jax.experimental import pallas as pl
from jax.experimental.pallas import tpu as pltpu
from jax.experimental.pallas import tpu_sc as plsc

NUM_USERS = 25000
NUM_ITEMS = 25000
N = NUM_USERS + NUM_ITEMS
E = 1600000
D = 32
B = 4096

NC = 2
NS = 16
NW = NC * NS

CK = 250
CHUNKS = E // CK
TILE_CHUNKS = CHUNKS // NW
SUP = 8
SUPERS = TILE_CHUNKS // SUP
PAIRS = SUP // 2

_FULL = CK // 16
_REM = CK - _FULL * 16
_WGROUPS = tuple((i * 16, tuple(range(16))) for i in range(_FULL))
if _REM:
    _WGROUPS += ((CK - 16, tuple(range(16 - _REM, 16))),)

SROWS = 250
NSTRIPES = N // SROWS
STRIPE_STEPS = (NSTRIPES + NS - 1) // NS

_PERM = np.arange(D).reshape(2, D // 2).T.reshape(-1)

_ILV = plsc.PackFormat.INTERLEAVED

_mesh = plsc.VectorSubcoreMesh(core_axis_name="c", subcore_axis_name="s")
_params = pltpu.CompilerParams(use_tc_tiling_on_sc=False, needs_layout_passes=False)


@functools.partial(
    pl.kernel,
    out_type=jax.ShapeDtypeStruct((NC, N, D), jnp.float32),
    mesh=_mesh,
    compiler_params=_params,
    scratch_types=[
        pltpu.VMEM_SHARED((N, D), jnp.float32),
        pltpu.VMEM((SUP, CK), jnp.int32),
        pltpu.VMEM((SUP, CK), jnp.int32),
        pltpu.VMEM((SUP, CK), jnp.float32),
        pltpu.VMEM((CK, D), jnp.bfloat16),
        pltpu.VMEM((CK, D), jnp.bfloat16),
        pltpu.VMEM((CK, D), jnp.float32),
        pltpu.VMEM((CK, D), jnp.float32),
        pltpu.SemaphoreType.DMA,
        pltpu.SemaphoreType.DMA,
        pltpu.SemaphoreType.DMA,
        pltpu.SemaphoreType.DMA,
    ],
)
def _spmv(table, src2d, dst2d, w2d, out, acc, srcb, dstb, wb, g0, g1, sc0, sc1,
          sem0, sem1, ssem0, ssem1):
    cid = lax.axis_index("c")
    sid = lax.axis_index("s")
    tile = cid * NS + sid

    zeros16 = jnp.zeros((16,), jnp.float32)

    def _zfill(r, _):
        sc0[r, pl.ds(0, 16)] = zeros16
        sc0[r, pl.ds(16, 16)] = zeros16
        return _

    lax.fori_loop(0, CK, _zfill, 0)

    def _zcopy(k, carry):
        c = sid + k * NS

        @pl.when(c < NSTRIPES)
        def _zc():
            pltpu.sync_copy(sc0, acc.at[pl.ds(c * SROWS, SROWS)])

        return carry

    lax.fori_loop(0, STRIPE_STEPS, _zcopy, 0)
    plsc.subcore_barrier()

    chunk0 = tile * TILE_CHUNKS

    def _scale(g, sc, c):
        for goff, glanes in _WGROUPS:
            wvec = wb[c, pl.ds(goff, 16)]
            for lane in glanes:
                e = goff + lane
                lo, hi = plsc.unpack(g[e], format=_ILV)
                wv = wvec[lane]
                sc[e, pl.ds(0, 16)] = lo * wv
                sc[e, pl.ds(16, 16)] = hi * wv

    def _super(s, _):
        base = chunk0 + s * SUP
        pltpu.sync_copy(src2d.at[pl.ds(base, SUP)], srcb)
        pltpu.sync_copy(dst2d.at[pl.ds(base, SUP)], dstb)
        pltpu.sync_copy(w2d.at[pl.ds(base, SUP)], wb)

        def _pair(pc, carry):
            c0 = pc * 2

            @pl.when(pc == 0)
            def _prime():
                pltpu.async_copy(table.at[srcb.at[0]], g0, sem0)

            pltpu.make_async_copy(table.at[pl.ds(0, CK)], g0, sem0).wait()
            pltpu.async_copy(table.at[srcb.at[c0 + 1]], g1, sem1)

            @pl.when(pc > 0)
            def _sc0_free():
                pltpu.make_async_copy(sc0, acc.at[pl.ds(0, CK)], ssem0).wait()

            _scale(g0, sc0, c0)
            pltpu.async_copy(sc0, acc.at[dstb.at[c0]], ssem0, add=True)
            pltpu.make_async_copy(table.at[pl.ds(0, CK)], g1, sem1).wait()

            @pl.when(pc + 1 < PAIRS)
            def _next():
                pltpu.async_copy(table.at[srcb.at[c0 + 2]], g0, sem0)

            @pl.when(pc > 0)
            def _sc1_free():
                pltpu.make_async_copy(sc1, acc.at[pl.ds(0, CK)], ssem1).wait()

            _scale(g1, sc1, c0 + 1)
            pltpu.async_copy(sc1, acc.at[dstb.at[c0 + 1]], ssem1, add=True)
            return carry

        lax.fori_loop(0, PAIRS, _pair, 0)
        pltpu.make_async_copy(sc0, acc.at[pl.ds(0, CK)], ssem0).wait()
        pltpu.make_async_copy(sc1, acc.at[pl.ds(0, CK)], ssem1).wait()
        return _

    lax.fori_loop(0, SUPERS, _super, 0)
    plsc.subcore_barrier()

    def _wcopy(k, carry):
        c = sid + k * NS

        @pl.when(c < NSTRIPES)
        def _wc():
            pltpu.sync_copy(acc.at[pl.ds(c * SROWS, SROWS)],
                            out.at[cid, pl.ds(c * SROWS, SROWS)])

        return carry

    lax.fori_loop(0, STRIPE_STEPS, _wcopy, 0)


CROWS = 400
CCHUNKS = N // CROWS
CSTEPS = (CCHUNKS + NW - 1) // NW


@functools.partial(
    pl.kernel,
    out_type=jax.ShapeDtypeStruct((N, D), jnp.bfloat16),
    mesh=_mesh,
    compiler_params=_params,
    scratch_types=[
        pltpu.VMEM((CROWS, D), jnp.float32),
        pltpu.VMEM((CROWS, D), jnp.float32),
        pltpu.VMEM((CROWS, D), jnp.bfloat16),
    ],
)
def _combine(p, out, a, b, o):
    cid = lax.axis_index("c")
    sid = lax.axis_index("s")
    tile = cid * NS + sid

    def _step(k, carry):
        c = tile + k * NW

        @pl.when(c < CCHUNKS)
        def _body():
            r0 = c * CROWS
            pltpu.sync_copy(p.at[0, pl.ds(r0, CROWS)], a)
            pltpu.sync_copy(p.at[1, pl.ds(r0, CROWS)], b)

            def _add(r, _):
                lo = a[r, pl.ds(0, 16)] + b[r, pl.ds(0, 16)]
                hi = a[r, pl.ds(16, 16)] + b[r, pl.ds(16, 16)]
                o[r] = plsc.pack(lo, hi, format=_ILV)
                return _

            lax.fori_loop(0, CROWS, _add, 0)
            pltpu.sync_copy(o, out.at[pl.ds(r0, CROWS)])

        return carry

    lax.fori_loop(0, CSTEPS, _step, 0)


PPT = B // NW


@functools.partial(
    pl.kernel,
    out_type=jax.ShapeDtypeStruct((B,), jnp.float32),
    mesh=_mesh,
    compiler_params=_params,
    scratch_types=[
        pltpu.VMEM((PPT,), jnp.int32),
        pltpu.VMEM((PPT,), jnp.int32),
        pltpu.VMEM((PPT,), jnp.int32),
        pltpu.VMEM((PPT, D), jnp.float32),
        pltpu.VMEM((PPT, D), jnp.float32),
        pltpu.VMEM((2, PPT, D), jnp.bfloat16),
        pltpu.VMEM((2, PPT, D), jnp.bfloat16),
        pltpu.VMEM((2, PPT, D), jnp.float32),
        pltpu.VMEM((2, PPT, D), jnp.float32),
        pltpu.VMEM((PPT,), jnp.float32),
        pltpu.SemaphoreType.DMA,
    ],
)
def _final(users, items, uemb, iemb, t1, t2, p3, out,
           uidx, iidx, iidx2, gu, gi, tu, ti, pu, pi, obuf, sem):
    cid = lax.axis_index("c")
    sid = lax.axis_index("s")
    tile = cid * NS + sid
    base = tile * PPT

    pltpu.sync_copy(users.at[pl.ds(base, PPT)], uidx)
    pltpu.sync_copy(items.at[pl.ds(base, PPT)], iidx)

    off = jnp.full((16,), NUM_USERS, jnp.int32)

    def _shift(v, _):
        iidx2[pl.ds(v * 16, 16)] = iidx[pl.ds(v * 16, 16)] + off
        return _

    lax.fori_loop(0, PPT // 16, _shift, 0)

    pltpu.async_copy(uemb.at[uidx], gu, sem).wait()
    pltpu.async_copy(iemb.at[iidx], gi, sem).wait()
    pltpu.async_copy(t1.at[uidx], tu.at[0], sem).wait()
    pltpu.async_copy(t2.at[uidx], tu.at[1], sem).wait()
    pltpu.async_copy(t1.at[iidx2], ti.at[0], sem).wait()
    pltpu.async_copy(t2.at[iidx2], ti.at[1], sem).wait()
    pltpu.async_copy(p3.at[0].at[uidx], pu.at[0], sem).wait()
    pltpu.async_copy(p3.at[1].at[uidx], pu.at[1], sem).wait()
    pltpu.async_copy(p3.at[0].at[iidx2], pi.at[0], sem).wait()
    pltpu.async_copy(p3.at[1].at[iidx2], pi.at[1], sem).wait()

    lane_iota = lax.iota(jnp.int32, 16)

    def _group(gp, carry):
        res = jnp.zeros((16,), jnp.float32)
        for lane in range(16):
            p_ = gp * 16 + lane
            su_lo = gu[p_, pl.ds(0, 16)] + pu[0, p_, pl.ds(0, 16)] + pu[1, p_, pl.ds(0, 16)]
            su_hi = gu[p_, pl.ds(16, 16)] + pu[0, p_, pl.ds(16, 16)] + pu[1, p_, pl.ds(16, 16)]
            si_lo = gi[p_, pl.ds(0, 16)] + pi[0, p_, pl.ds(0, 16)] + pi[1, p_, pl.ds(0, 16)]
            si_hi = gi[p_, pl.ds(16, 16)] + pi[0, p_, pl.ds(16, 16)] + pi[1, p_, pl.ds(16, 16)]
            for k in range(2):
                ulo, uhi = plsc.unpack(tu[k, p_], format=_ILV)
                ilo, ihi = plsc.unpack(ti[k, p_], format=_ILV)
                su_lo = su_lo + ulo
                su_hi = su_hi + uhi
                si_lo = si_lo + ilo
                si_hi = si_hi + ihi
            s = jnp.sum(su_lo * si_lo + su_hi * si_hi) * 0.0625
            res = jnp.where(lane_iota == lane, s, res)
        obuf[pl.ds(gp * 16, 16)] = res
        return carry

    lax.fori_loop(0, PPT // 16, _group, 0)
    pltpu.sync_copy(obuf, out.at[pl.ds(base, PPT)])


def kernel(users, items, edge_index, edge_weight, user_emb, item_emb):
    src2d = edge_index[0].reshape(CHUNKS, CK)
    dst2d = edge_index[1].reshape(CHUNKS, CK)
    w2d = edge_weight.reshape(CHUNKS, CK)
    tab0 = jnp.concatenate([user_emb, item_emb], axis=0)
    tab0 = tab0[:, _PERM].astype(jnp.bfloat16)

    p1 = _spmv(tab0, src2d, dst2d, w2d)
    t1 = _combine(p1)
    p2 = _spmv(t1, src2d, dst2d, w2d)
    t2 = _combine(p2)
    p3 = _spmv(t2, src2d, dst2d, w2d)
    return _final(users, items, user_emb, item_emb, t1, t2, p3)


# speedup vs baseline: 1.5806x; 1.0312x over previous
import functools

import jax
import jax.numpy as jnp
import numpy as np
from jax import lax
from jax.experimental import pallas as pl
from jax.experimental.pallas import tpu as pltpu
from jax.experimental.pallas import tpu_sc as plsc

NUM_USERS = 25000
NUM_ITEMS = 25000
N = NUM_USERS + NUM_ITEMS
E = 1600000
D = 32
B = 4096

NC = 2
NS = 16
NW = NC * NS

CK = 250
CHUNKS = E // CK
TILE_CHUNKS = CHUNKS // NW
SUP = 10
SUPERS = TILE_CHUNKS // SUP
PAIRS = SUP // 2

_FULL = CK // 16
_REM = CK - _FULL * 16
_WGROUPS = tuple((i * 16, tuple(range(16))) for i in range(_FULL))
if _REM:
    _WGROUPS += ((CK - 16, tuple(range(16 - _REM, 16))),)

SROWS = 200
NSTRIPES = N // SROWS
STRIPE_STEPS = (NSTRIPES + NS - 1) // NS

_PERM = np.arange(D).reshape(2, D // 2).T.reshape(-1)

_ILV = plsc.PackFormat.INTERLEAVED

_mesh = plsc.VectorSubcoreMesh(core_axis_name="c", subcore_axis_name="s")
_params = pltpu.CompilerParams(use_tc_tiling_on_sc=False, needs_layout_passes=False)


@functools.partial(
    pl.kernel,
    out_type=jax.ShapeDtypeStruct((NC, N, D), jnp.float32),
    mesh=_mesh,
    compiler_params=_params,
    scratch_types=[
        pltpu.VMEM_SHARED((N, D), jnp.float32),
        pltpu.VMEM((SUP, CK), jnp.int32),
        pltpu.VMEM((SUP, CK), jnp.int32),
        pltpu.VMEM((SUP, CK), jnp.float32),
        pltpu.VMEM((CK, D), jnp.bfloat16),
        pltpu.VMEM((CK, D), jnp.bfloat16),
        pltpu.VMEM((CK, D), jnp.float32),
        pltpu.VMEM((SROWS, D), jnp.float32),
        pltpu.SemaphoreType.DMA,
        pltpu.SemaphoreType.DMA,
    ],
)
def _spmv(table, src2d, dst2d, w2d, out, acc, srcb, dstb, wb, g0, g1, sc0,
          zbuf, sem0, sem1):
    cid = lax.axis_index("c")
    sid = lax.axis_index("s")
    tile = cid * NS + sid

    zeros16 = jnp.zeros((16,), jnp.float32)

    def _zfill(r, _):
        zbuf[r, pl.ds(0, 16)] = zeros16
        zbuf[r, pl.ds(16, 16)] = zeros16
        return _

    lax.fori_loop(0, SROWS, _zfill, 0)

    def _zcopy(k, carry):
        c = sid + k * NS

        @pl.when(c < NSTRIPES)
        def _zc():
            pltpu.sync_copy(zbuf, acc.at[pl.ds(c * SROWS, SROWS)])

        return carry

    lax.fori_loop(0, STRIPE_STEPS, _zcopy, 0)
    plsc.subcore_barrier()

    chunk0 = tile * TILE_CHUNKS

    def _scale(g, sc, c):
        for goff, glanes in _WGROUPS:
            wvec = wb[c, pl.ds(goff, 16)]
            for lane in glanes:
                e = goff + lane
                lo, hi = plsc.unpack(g[e], format=_ILV)
                wv = wvec[lane]
                sc[e, pl.ds(0, 16)] = lo * wv
                sc[e, pl.ds(16, 16)] = hi * wv

    def _super(s, _):
        base = chunk0 + s * SUP
        pltpu.sync_copy(src2d.at[pl.ds(base, SUP)], srcb)
        pltpu.sync_copy(dst2d.at[pl.ds(base, SUP)], dstb)
        pltpu.sync_copy(w2d.at[pl.ds(base, SUP)], wb)

        def _pair(pc, carry):
            c0 = pc * 2

            @pl.when(pc == 0)
            def _prime():
                pltpu.async_copy(table.at[srcb.at[0]], g0, sem0)

            pltpu.make_async_copy(table.at[pl.ds(0, CK)], g0, sem0).wait()
            pltpu.async_copy(table.at[srcb.at[c0 + 1]], g1, sem1)
            _scale(g0, sc0, c0)
            pltpu.sync_copy(sc0, acc.at[dstb.at[c0]], add=True)

            pltpu.make_async_copy(table.at[pl.ds(0, CK)], g1, sem1).wait()

            @pl.when(pc + 1 < PAIRS)
            def _next():
                pltpu.async_copy(table.at[srcb.at[c0 + 2]], g0, sem0)

            _scale(g1, sc0, c0 + 1)
            pltpu.sync_copy(sc0, acc.at[dstb.at[c0 + 1]], add=True)
            return carry

        lax.fori_loop(0, PAIRS, _pair, 0)
        return _

    lax.fori_loop(0, SUPERS, _super, 0)
    plsc.subcore_barrier()

    def _wcopy(k, carry):
        c = sid + k * NS

        @pl.when(c < NSTRIPES)
        def _wc():
            pltpu.sync_copy(acc.at[pl.ds(c * SROWS, SROWS)],
                            out.at[cid, pl.ds(c * SROWS, SROWS)])

        return carry

    lax.fori_loop(0, STRIPE_STEPS, _wcopy, 0)


CROWS = 400
CCHUNKS = N // CROWS
CSTEPS = (CCHUNKS + NW - 1) // NW


@functools.partial(
    pl.kernel,
    out_type=jax.ShapeDtypeStruct((N, D), jnp.bfloat16),
    mesh=_mesh,
    compiler_params=_params,
    scratch_types=[
        pltpu.VMEM((CROWS, D), jnp.float32),
        pltpu.VMEM((CROWS, D), jnp.float32),
        pltpu.VMEM((CROWS, D), jnp.bfloat16),
    ],
)
def _combine(p, out, a, b, o):
    cid = lax.axis_index("c")
    sid = lax.axis_index("s")
    tile = cid * NS + sid

    def _step(k, carry):
        c = tile + k * NW

        @pl.when(c < CCHUNKS)
        def _body():
            r0 = c * CROWS
            pltpu.sync_copy(p.at[0, pl.ds(r0, CROWS)], a)
            pltpu.sync_copy(p.at[1, pl.ds(r0, CROWS)], b)

            def _add(r, _):
                lo = a[r, pl.ds(0, 16)] + b[r, pl.ds(0, 16)]
                hi = a[r, pl.ds(16, 16)] + b[r, pl.ds(16, 16)]
                o[r] = plsc.pack(lo, hi, format=_ILV)
                return _

            lax.fori_loop(0, CROWS, _add, 0)
            pltpu.sync_copy(o, out.at[pl.ds(r0, CROWS)])

        return carry

    lax.fori_loop(0, CSTEPS, _step, 0)


PPT = B // NW


@functools.partial(
    pl.kernel,
    out_type=jax.ShapeDtypeStruct((B,), jnp.float32),
    mesh=_mesh,
    compiler_params=_params,
    scratch_types=[
        pltpu.VMEM((PPT,), jnp.int32),
        pltpu.VMEM((PPT,), jnp.int32),
        pltpu.VMEM((PPT,), jnp.int32),
        pltpu.VMEM((PPT, D), jnp.float32),
        pltpu.VMEM((PPT, D), jnp.float32),
        pltpu.VMEM((2, PPT, D), jnp.bfloat16),
        pltpu.VMEM((2, PPT, D), jnp.bfloat16),
        pltpu.VMEM((2, PPT, D), jnp.float32),
        pltpu.VMEM((2, PPT, D), jnp.float32),
        pltpu.VMEM((PPT,), jnp.float32),
        pltpu.SemaphoreType.DMA,
    ],
)
def _final(users, items, uemb, iemb, t1, t2, p3, out,
           uidx, iidx, iidx2, gu, gi, tu, ti, pu, pi, obuf, sem):
    cid = lax.axis_index("c")
    sid = lax.axis_index("s")
    tile = cid * NS + sid
    base = tile * PPT

    pltpu.sync_copy(users.at[pl.ds(base, PPT)], uidx)
    pltpu.sync_copy(items.at[pl.ds(base, PPT)], iidx)

    off = jnp.full((16,), NUM_USERS, jnp.int32)

    def _shift(v, _):
        iidx2[pl.ds(v * 16, 16)] = iidx[pl.ds(v * 16, 16)] + off
        return _

    lax.fori_loop(0, PPT // 16, _shift, 0)

    pltpu.async_copy(uemb.at[uidx], gu, sem).wait()
    pltpu.async_copy(iemb.at[iidx], gi, sem).wait()
    pltpu.async_copy(t1.at[uidx], tu.at[0], sem).wait()
    pltpu.async_copy(t2.at[uidx], tu.at[1], sem).wait()
    pltpu.async_copy(t1.at[iidx2], ti.at[0], sem).wait()
    pltpu.async_copy(t2.at[iidx2], ti.at[1], sem).wait()
    pltpu.async_copy(p3.at[0].at[uidx], pu.at[0], sem).wait()
    pltpu.async_copy(p3.at[1].at[uidx], pu.at[1], sem).wait()
    pltpu.async_copy(p3.at[0].at[iidx2], pi.at[0], sem).wait()
    pltpu.async_copy(p3.at[1].at[iidx2], pi.at[1], sem).wait()

    lane_iota = lax.iota(jnp.int32, 16)

    def _group(gp, carry):
        res = jnp.zeros((16,), jnp.float32)
        for lane in range(16):
            p_ = gp * 16 + lane
            su_lo = gu[p_, pl.ds(0, 16)] + pu[0, p_, pl.ds(0, 16)] + pu[1, p_, pl.ds(0, 16)]
            su_hi = gu[p_, pl.ds(16, 16)] + pu[0, p_, pl.ds(16, 16)] + pu[1, p_, pl.ds(16, 16)]
            si_lo = gi[p_, pl.ds(0, 16)] + pi[0, p_, pl.ds(0, 16)] + pi[1, p_, pl.ds(0, 16)]
            si_hi = gi[p_, pl.ds(16, 16)] + pi[0, p_, pl.ds(16, 16)] + pi[1, p_, pl.ds(16, 16)]
            for k in range(2):
                ulo, uhi = plsc.unpack(tu[k, p_], format=_ILV)
                ilo, ihi = plsc.unpack(ti[k, p_], format=_ILV)
                su_lo = su_lo + ulo
                su_hi = su_hi + uhi
                si_lo = si_lo + ilo
                si_hi = si_hi + ihi
            s = jnp.sum(su_lo * si_lo + su_hi * si_hi) * 0.0625
            res = jnp.where(lane_iota == lane, s, res)
        obuf[pl.ds(gp * 16, 16)] = res
        return carry

    lax.fori_loop(0, PPT // 16, _group, 0)
    pltpu.sync_copy(obuf, out.at[pl.ds(base, PPT)])


def kernel(users, items, edge_index, edge_weight, user_emb, item_emb):
    src2d = edge_index[0].reshape(CHUNKS, CK)
    dst2d = edge_index[1].reshape(CHUNKS, CK)
    w2d = edge_weight.reshape(CHUNKS, CK)
    tab0 = jnp.concatenate([user_emb, item_emb], axis=0)
    tab0 = tab0[:, _PERM].astype(jnp.bfloat16)

    p1 = _spmv(tab0, src2d, dst2d, w2d)
    t1 = _combine(p1)
    p2 = _spmv(t1, src2d, dst2d, w2d)
    t2 = _combine(p2)
    p3 = _spmv(t2, src2d, dst2d, w2d)
    return _final(users, items, user_emb, item_emb, t1, t2, p3)
